# initial kernel scaffold (unmeasured)
import jax
import jax.numpy as jnp
from jax import lax
from jax.experimental import pallas as pl
from jax.experimental.pallas import tpu as pltpu

N_DEV = 8
M = 1024
D = 1024
CH = M // N_DEV


def _mod(v):
    return lax.rem(v + 2 * N_DEV, N_DEV)


def kernel(partial, resid, gamma):
    gamma2 = gamma.reshape(1, D)

    def body(x_ref, resid_ref, gamma_ref, out_ref,
             send_buf, recv_bufs,
             rs_send_sems, rs_recv_sems, ag_send_sems, ag_recv_sems):
        p = lax.axis_index("i")
        right = _mod(p + 1)
        f = _mod(p + 1)

        def rows(idx):
            return pl.ds(idx * CH, CH)

        send_buf[:, :] = x_ref[0, rows(p), :]
        for s in range(N_DEV - 1):
            rdma = pltpu.make_async_remote_copy(
                src_ref=send_buf,
                dst_ref=recv_bufs.at[s],
                send_sem=rs_send_sems.at[s],
                recv_sem=rs_recv_sems.at[s],
                device_id=(right,),
                device_id_type=pl.DeviceIdType.MESH,
            )
            rdma.start()
            rdma.wait()
            if s < N_DEV - 2:
                nxt = _mod(p - s - 1)
                send_buf[:, :] = recv_bufs[s] + x_ref[0, rows(nxt), :]

        y = recv_bufs[N_DEV - 2] + x_ref[0, rows(f), :] + resid_ref[rows(f), :]
        rms = jnp.sqrt(jnp.mean(y * y, axis=-1, keepdims=True) + 1e-6)
        out_ref[rows(f), :] = y / rms * gamma_ref[:, :]

        for t in range(N_DEV - 1):
            idx = _mod(f - t)
            rdma = pltpu.make_async_remote_copy(
                src_ref=out_ref.at[rows(idx), :],
                dst_ref=out_ref.at[rows(idx), :],
                send_sem=ag_send_sems.at[t],
                recv_sem=ag_recv_sems.at[t],
                device_id=(right,),
                device_id_type=pl.DeviceIdType.MESH,
            )
            rdma.start()
            rdma.wait()

    return pl.pallas_call(
        body,
        out_shape=jax.ShapeDtypeStruct((M, D), jnp.float32),
        in_specs=[
            pl.BlockSpec(memory_space=pltpu.VMEM),
            pl.BlockSpec(memory_space=pltpu.VMEM),
            pl.BlockSpec(memory_space=pltpu.VMEM),
        ],
        out_specs=pl.BlockSpec(memory_space=pltpu.VMEM),
        scratch_shapes=[
            pltpu.VMEM((CH, D), jnp.float32),
            pltpu.VMEM((N_DEV - 1, CH, D), jnp.float32),
            pltpu.SemaphoreType.DMA((N_DEV - 1,)),
            pltpu.SemaphoreType.DMA((N_DEV - 1,)),
            pltpu.SemaphoreType.DMA((N_DEV - 1,)),
            pltpu.SemaphoreType.DMA((N_DEV - 1,)),
        ],
        compiler_params=pltpu.CompilerParams(collective_id=0),
    )(partial, resid, gamma2)


# baseline (device time: 118735 ns/iter reference)
import jax
import jax.numpy as jnp
from jax import lax
from jax.experimental import pallas as pl
from jax.experimental.pallas import tpu as pltpu

N_DEV = 8
M = 1024
D = 1024
CH = M // N_DEV


def _mod(v):
    return lax.rem(v + 2 * N_DEV, N_DEV)


def kernel(partial, resid, gamma):
    gamma2 = gamma.reshape(1, D)

    def body(x_ref, resid_ref, gamma_ref, out_ref,
             send_buf, recv_bufs,
             rs_send_sems, rs_recv_sems, ag_send_sems, ag_recv_sems):
        p = lax.axis_index("i")
        right = _mod(p + 1)
        f = _mod(p + 1)

        def rows(idx):
            return pl.ds(idx * CH, CH)

        send_buf[:, :] = x_ref[0, rows(p), :]
        for s in range(N_DEV - 1):
            rdma = pltpu.make_async_remote_copy(
                src_ref=send_buf,
                dst_ref=recv_bufs.at[s],
                send_sem=rs_send_sems.at[s],
                recv_sem=rs_recv_sems.at[s],
                device_id=(right,),
                device_id_type=pl.DeviceIdType.MESH,
            )
            rdma.start()
            rdma.wait()
            if s < N_DEV - 2:
                nxt = _mod(p - s - 1)
                send_buf[:, :] = recv_bufs[s] + x_ref[0, rows(nxt), :]

        y = recv_bufs[N_DEV - 2] + x_ref[0, rows(f), :] + resid_ref[rows(f), :]
        rms = jnp.sqrt(jnp.mean(y * y, axis=-1, keepdims=True) + 1e-6)
        out_ref[rows(f), :] = y / rms * gamma_ref[:, :]

        for t in range(N_DEV - 1):
            idx = _mod(f - t)
            rdma = pltpu.make_async_remote_copy(
                src_ref=out_ref.at[rows(idx), :],
                dst_ref=out_ref.at[rows(idx), :],
                send_sem=ag_send_sems.at[t],
                recv_sem=ag_recv_sems.at[t],
                device_id=(right,),
                device_id_type=pl.DeviceIdType.MESH,
            )
            rdma.start()
            rdma.wait()

    return pl.pallas_call(
        body,
        out_shape=jax.ShapeDtypeStruct((M, D), jnp.float32),
        in_specs=[
            pl.BlockSpec(memory_space=pltpu.VMEM),
            pl.BlockSpec(memory_space=pltpu.VMEM),
            pl.BlockSpec(memory_space=pltpu.VMEM),
        ],
        out_specs=pl.BlockSpec(memory_space=pltpu.VMEM),
        scratch_shapes=[
            pltpu.VMEM((CH, D), jnp.float32),
            pltpu.VMEM((N_DEV - 1, CH, D), jnp.float32),
            pltpu.SemaphoreType.DMA((N_DEV - 1,)),
            pltpu.SemaphoreType.DMA((N_DEV - 1,)),
            pltpu.SemaphoreType.DMA((N_DEV - 1,)),
            pltpu.SemaphoreType.DMA((N_DEV - 1,)),
        ],
    )(partial, resid, gamma2)


# device time: 54332 ns/iter; 2.1854x vs baseline; 2.1854x over previous
import jax
import jax.numpy as jnp
from jax import lax
from jax.experimental import pallas as pl
from jax.experimental.pallas import tpu as pltpu

N_DEV = 8
M = 1024
D = 1024

PART_SIZES = (384, 320, 320)
PART_BASES = (0, 384, 704)
AXIS_ORDERS = (("x", "y", "z"), ("y", "z", "x"), ("z", "x", "y"))
RB_OFFS = tuple((0, m // 2, 3 * m // 4) for m in PART_SIZES)


def _partner_and_bit(p, axis):
    q = lax.rem(p, 4)
    zc = lax.div(p, 4)
    if axis == "x":
        return zc * 4 + jnp.bitwise_xor(q, 1), jnp.bitwise_and(
            jnp.bitwise_xor(q, lax.div(q, 2)), 1
        )
    if axis == "y":
        return zc * 4 + (3 - q), lax.div(q, 2)
    return jnp.bitwise_xor(p, 4), zc


def kernel(partial, resid, gamma):
    gamma2 = gamma.reshape(1, D)

    def body(x_ref, resid_ref, gamma_ref, out_ref,
             acc, rb0, rb1, rb2,
             rs_send, rs_recv, ag_send, ag_recv):
        p = lax.axis_index("i")
        rbufs = (rb0, rb1, rb2)

        pb = [[_partner_and_bit(p, AXIS_ORDERS[k][j]) for j in range(3)]
              for k in range(3)]

        offs = [jnp.int32(b) for b in PART_BASES]
        for j in range(3):
            rdmas = []
            for k in range(3):
                half = PART_SIZES[k] >> (j + 1)
                partner, b = pb[k][j]
                send_off = offs[k] + (1 - b) * half
                if j == 0:
                    src = x_ref.at[0, pl.ds(send_off, half), :]
                else:
                    src = acc.at[pl.ds(send_off, half), :]
                rdma = pltpu.make_async_remote_copy(
                    src_ref=src,
                    dst_ref=rbufs[k].at[pl.ds(RB_OFFS[k][j], half), :],
                    send_sem=rs_send.at[k, j],
                    recv_sem=rs_recv.at[k, j],
                    device_id=(partner,),
                    device_id_type=pl.DeviceIdType.MESH,
                )
                rdma.start()
                rdmas.append(rdma)
            for k in range(3):
                half = PART_SIZES[k] >> (j + 1)
                _, b = pb[k][j]
                rdmas[k].wait()
                keep = pl.ds(offs[k] + b * half, half)
                recv = rbufs[k][pl.ds(RB_OFFS[k][j], half), :]
                if j == 0:
                    acc[keep, :] = x_ref[0, keep, :] + recv
                else:
                    acc[keep, :] = acc[keep, :] + recv
                offs[k] = offs[k] + b * half

        for k in range(3):
            blk = PART_SIZES[k] >> 3
            own = pl.ds(offs[k], blk)
            y = acc[own, :] + resid_ref[own, :]
            rms = jnp.sqrt(jnp.mean(y * y, axis=-1, keepdims=True) + 1e-6)
            out_ref[own, :] = y / rms * gamma_ref[:, :]

        for j in (2, 1, 0):
            rdmas = []
            for k in range(3):
                size = PART_SIZES[k] >> (j + 1)
                partner, b = pb[k][j]
                blk = pl.ds(offs[k], size)
                rdma = pltpu.make_async_remote_copy(
                    src_ref=out_ref.at[blk, :],
                    dst_ref=out_ref.at[blk, :],
                    send_sem=ag_send.at[k, j],
                    recv_sem=ag_recv.at[k, j],
                    device_id=(partner,),
                    device_id_type=pl.DeviceIdType.MESH,
                )
                rdma.start()
                rdmas.append(rdma)
            for k in range(3):
                size = PART_SIZES[k] >> (j + 1)
                _, b = pb[k][j]
                rdmas[k].wait()
                offs[k] = offs[k] - b * size

    return pl.pallas_call(
        body,
        out_shape=jax.ShapeDtypeStruct((M, D), jnp.float32),
        in_specs=[
            pl.BlockSpec(memory_space=pltpu.VMEM),
            pl.BlockSpec(memory_space=pltpu.VMEM),
            pl.BlockSpec(memory_space=pltpu.VMEM),
        ],
        out_specs=pl.BlockSpec(memory_space=pltpu.VMEM),
        scratch_shapes=[
            pltpu.VMEM((M, D), jnp.float32),
            pltpu.VMEM((7 * PART_SIZES[0] // 8, D), jnp.float32),
            pltpu.VMEM((7 * PART_SIZES[1] // 8, D), jnp.float32),
            pltpu.VMEM((7 * PART_SIZES[2] // 8, D), jnp.float32),
            pltpu.SemaphoreType.DMA((3, 3)),
            pltpu.SemaphoreType.DMA((3, 3)),
            pltpu.SemaphoreType.DMA((3, 3)),
            pltpu.SemaphoreType.DMA((3, 3)),
        ],
    )(partial, resid, gamma2)


# device time: 50888 ns/iter; 2.3333x vs baseline; 1.0677x over previous
import jax
import jax.numpy as jnp
from jax import lax
from jax.experimental import pallas as pl
from jax.experimental.pallas import tpu as pltpu

N_DEV = 8
M = 1024
D = 1024

PART_SIZES = (384, 320, 320)
PART_BASES = (0, 384, 704)
AXIS_ORDERS = (("x", "y", "z"), ("y", "z", "x"), ("z", "x", "y"))
RB_OFFS = tuple((0, m // 2, 3 * m // 4) for m in PART_SIZES)
ORDER = (1, 2, 0)


def _partner_and_bit(p, axis):
    q = lax.rem(p, 4)
    zc = lax.div(p, 4)
    if axis == "x":
        return zc * 4 + jnp.bitwise_xor(q, 1), jnp.bitwise_and(
            jnp.bitwise_xor(q, lax.div(q, 2)), 1
        )
    if axis == "y":
        return zc * 4 + (3 - q), lax.div(q, 2)
    return jnp.bitwise_xor(p, 4), zc


def kernel(partial, resid, gamma):
    gamma2 = gamma.reshape(1, D)

    def body(x_ref, resid_ref, gamma_ref, out_ref,
             acc, rb0, rb1, rb2, res_loc,
             rs_send, rs_recv, ag_send, ag_recv, res_sems):
        p = lax.axis_index("i")
        rbufs = (rb0, rb1, rb2)

        pb = [[_partner_and_bit(p, AXIS_ORDERS[k][j]) for j in range(3)]
              for k in range(3)]

        barrier_sem = pltpu.get_barrier_semaphore()
        nbrs = [_partner_and_bit(p, a)[0] for a in ("x", "y", "z")]
        for nbr in nbrs:
            pl.semaphore_signal(
                barrier_sem, inc=1,
                device_id=(nbr,), device_id_type=pl.DeviceIdType.MESH,
            )
        pl.semaphore_wait(barrier_sem, 3)

        offs_fin = []
        res_copies = []
        for k in range(3):
            m = PART_SIZES[k]
            b0, b1, b2 = pb[k][0][1], pb[k][1][1], pb[k][2][1]
            off = PART_BASES[k] + b0 * (m >> 1) + b1 * (m >> 2) + b2 * (m >> 3)
            offs_fin.append(off)
            cp = pltpu.make_async_copy(
                resid_ref.at[pl.ds(off, m >> 3), :],
                res_loc.at[pl.ds(RES_OFFS[k], m >> 3), :],
                res_sems.at[k],
            )
            cp.start()
            res_copies.append(cp)

        def rs_rdma(k, j, offs):
            half = PART_SIZES[k] >> (j + 1)
            partner, b = pb[k][j]
            send_off = offs[k] + (1 - b) * half
            if j == 0:
                src = x_ref.at[0, pl.ds(send_off, half), :]
            else:
                src = acc.at[pl.ds(send_off, half), :]
            return pltpu.make_async_remote_copy(
                src_ref=src,
                dst_ref=rbufs[k].at[pl.ds(RB_OFFS[k][j], half), :],
                send_sem=rs_send.at[k, j],
                recv_sem=rs_recv.at[k, j],
                device_id=(partner,),
                device_id_type=pl.DeviceIdType.MESH,
            )

        def ag_rdma(k, j, offs):
            size = PART_SIZES[k] >> (j + 1)
            partner, _ = pb[k][j]
            blk = pl.ds(offs[k], size)
            return pltpu.make_async_remote_copy(
                src_ref=out_ref.at[blk, :],
                dst_ref=out_ref.at[blk, :],
                send_sem=ag_send.at[k, j],
                recv_sem=ag_recv.at[k, j],
                device_id=(partner,),
                device_id_type=pl.DeviceIdType.MESH,
            )

        offs = [jnp.int32(b) for b in PART_BASES]
        rdmas = {}
        for k in ORDER:
            rdmas[k] = rs_rdma(k, 0, offs)
            rdmas[k].start()
        for j in range(3):
            for k in ORDER:
                half = PART_SIZES[k] >> (j + 1)
                _, b = pb[k][j]
                rdmas[k].wait()
                keep = pl.ds(offs[k] + b * half, half)
                recv = rbufs[k][pl.ds(RB_OFFS[k][j], half), :]
                if j == 0:
                    acc[keep, :] = x_ref[0, keep, :] + recv
                else:
                    acc[keep, :] = acc[keep, :] + recv
                offs[k] = offs[k] + b * half
                if j < 2:
                    rdmas[k] = rs_rdma(k, j + 1, offs)
                    rdmas[k].start()
                else:
                    blk = PART_SIZES[k] >> 3
                    own = pl.ds(offs[k], blk)
                    res_copies[k].wait()
                    y = acc[own, :] + res_loc[pl.ds(RES_OFFS[k], blk), :]
                    rms = jnp.sqrt(
                        jnp.mean(y * y, axis=-1, keepdims=True) + 1e-6
                    )
                    out_ref[own, :] = y / rms * gamma_ref[:, :]
                    rdmas[k] = ag_rdma(k, 2, offs)
                    rdmas[k].start()

        for j in (2, 1, 0):
            for k in ORDER:
                size = PART_SIZES[k] >> (j + 1)
                _, b = pb[k][j]
                rdmas[k].wait()
                offs[k] = offs[k] - b * size
                if j > 0:
                    rdmas[k] = ag_rdma(k, j - 1, offs)
                    rdmas[k].start()

    return pl.pallas_call(
        body,
        out_shape=jax.ShapeDtypeStruct((M, D), jnp.float32),
        in_specs=[
            pl.BlockSpec(memory_space=pltpu.VMEM),
            pl.BlockSpec(memory_space=pl.ANY),
            pl.BlockSpec(memory_space=pltpu.VMEM),
        ],
        out_specs=pl.BlockSpec(memory_space=pltpu.VMEM),
        scratch_shapes=[
            pltpu.VMEM((M, D), jnp.float32),
            pltpu.VMEM((7 * PART_SIZES[0] // 8, D), jnp.float32),
            pltpu.VMEM((7 * PART_SIZES[1] // 8, D), jnp.float32),
            pltpu.VMEM((7 * PART_SIZES[2] // 8, D), jnp.float32),
            pltpu.VMEM((M // 8, D), jnp.float32),
            pltpu.SemaphoreType.DMA((3, 3)),
            pltpu.SemaphoreType.DMA((3, 3)),
            pltpu.SemaphoreType.DMA((3, 3)),
            pltpu.SemaphoreType.DMA((3, 3)),
            pltpu.SemaphoreType.DMA((3,)),
        ],
        compiler_params=pltpu.CompilerParams(collective_id=0),
    )(partial, resid, gamma2)


RES_OFFS = (0, PART_SIZES[0] // 8, PART_SIZES[0] // 8 + PART_SIZES[1] // 8)


# device time: 43815 ns/iter; 2.7099x vs baseline; 1.1614x over previous
import jax
import jax.numpy as jnp
from jax import lax
from jax.experimental import pallas as pl
from jax.experimental.pallas import tpu as pltpu

N_DEV = 8
M = 1024
D = 1024

SIZES = (192, 160, 192, 160, 160, 160)
BASES = (0, 192, 352, 544, 704, 864)
ORDERS = (
    ("x", "y", "z"), ("x", "z", "y"),
    ("y", "z", "x"), ("y", "x", "z"),
    ("z", "x", "y"), ("z", "y", "x"),
)
RB_OFFS = tuple((0, s // 2, 3 * s // 4) for s in SIZES)
RES_OFFS = (0, 48, 88, 136, 176, 216)
ORDER = (1, 3, 4, 5, 0, 2)


def _partner_and_bit(p, axis):
    q = lax.rem(p, 4)
    zc = lax.div(p, 4)
    if axis == "x":
        return zc * 4 + jnp.bitwise_xor(q, 1), jnp.bitwise_and(
            jnp.bitwise_xor(q, lax.div(q, 2)), 1
        )
    if axis == "y":
        return zc * 4 + (3 - q), lax.div(q, 2)
    return jnp.bitwise_xor(p, 4), zc


def kernel(partial, resid, gamma):
    gamma2 = gamma.reshape(1, D)

    def body(x_ref, resid_ref, gamma_ref, out_ref,
             acc, rb0, rb1, rb2, rb3, rb4, rb5, res_loc,
             send_sems, recv_sems, res_sems):
        p = lax.axis_index("i")
        rbufs = (rb0, rb1, rb2, rb3, rb4, rb5)

        pb = [[_partner_and_bit(p, ORDERS[c][j]) for j in range(3)]
              for c in range(6)]

        barrier_sem = pltpu.get_barrier_semaphore()
        for axis in ("x", "y", "z"):
            pl.semaphore_signal(
                barrier_sem, inc=1,
                device_id=(_partner_and_bit(p, axis)[0],),
                device_id_type=pl.DeviceIdType.MESH,
            )
        pl.semaphore_wait(barrier_sem, 3)

        off_fin = []
        res_copies = []
        for c in range(6):
            s = SIZES[c]
            b0, b1 = pb[c][0][1], pb[c][1][1]
            off = BASES[c] + b0 * (s >> 1) + b1 * (s >> 2)
            off_fin.append(off)
            cp = pltpu.make_async_copy(
                resid_ref.at[pl.ds(off, s >> 2), :],
                res_loc.at[pl.ds(RES_OFFS[c], s >> 2), :],
                res_sems.at[c],
            )
            cp.start()
            res_copies.append(cp)

        def make_rdma(c, ph, offs):
            s = SIZES[c]
            partner = pb[c][(0, 1, 2, 1, 0)[ph]][0]
            if ph == 0:
                half = s >> 1
                b = pb[c][0][1]
                src = x_ref.at[0, pl.ds(offs[c] + (1 - b) * half, half), :]
                dst = rbufs[c].at[pl.ds(RB_OFFS[c][0], half), :]
            elif ph == 1:
                half = s >> 2
                b = pb[c][1][1]
                src = acc.at[pl.ds(offs[c] + (1 - b) * half, half), :]
                dst = rbufs[c].at[pl.ds(RB_OFFS[c][1], half), :]
            elif ph == 2:
                src = acc.at[pl.ds(offs[c], s >> 2), :]
                dst = rbufs[c].at[pl.ds(RB_OFFS[c][2], s >> 2), :]
            elif ph == 3:
                src = out_ref.at[pl.ds(offs[c], s >> 2), :]
                dst = out_ref.at[pl.ds(offs[c], s >> 2), :]
            else:
                src = out_ref.at[pl.ds(offs[c], s >> 1), :]
                dst = out_ref.at[pl.ds(offs[c], s >> 1), :]
            return pltpu.make_async_remote_copy(
                src_ref=src, dst_ref=dst,
                send_sem=send_sems.at[c, ph],
                recv_sem=recv_sems.at[c, ph],
                device_id=(partner,),
                device_id_type=pl.DeviceIdType.MESH,
            )

        offs = [jnp.int32(b) for b in BASES]
        rdmas = {}
        for c in ORDER:
            rdmas[c] = make_rdma(c, 0, offs)
            rdmas[c].start()

        for ph in range(4):
            for c in ORDER:
                s = SIZES[c]
                rdmas[c].wait()
                if ph == 0:
                    b = pb[c][0][1]
                    keep = pl.ds(offs[c] + b * (s >> 1), s >> 1)
                    acc[keep, :] = (
                        x_ref[0, keep, :]
                        + rbufs[c][pl.ds(RB_OFFS[c][0], s >> 1), :]
                    )
                    offs[c] = offs[c] + b * (s >> 1)
                elif ph == 1:
                    b = pb[c][1][1]
                    keep = pl.ds(offs[c] + b * (s >> 2), s >> 2)
                    acc[keep, :] = (
                        acc[keep, :]
                        + rbufs[c][pl.ds(RB_OFFS[c][1], s >> 2), :]
                    )
                    offs[c] = offs[c] + b * (s >> 2)
                elif ph == 2:
                    own = pl.ds(offs[c], s >> 2)
                    res_copies[c].wait()
                    y = (
                        acc[own, :]
                        + rbufs[c][pl.ds(RB_OFFS[c][2], s >> 2), :]
                        + res_loc[pl.ds(RES_OFFS[c], s >> 2), :]
                    )
                    rms = jnp.sqrt(
                        jnp.mean(y * y, axis=-1, keepdims=True) + 1e-6
                    )
                    out_ref[own, :] = y / rms * gamma_ref[:, :]
                else:
                    offs[c] = offs[c] - pb[c][1][1] * (s >> 2)
                rdmas[c] = make_rdma(c, ph + 1, offs)
                rdmas[c].start()

        for c in ORDER:
            rdmas[c].wait()

    return pl.pallas_call(
        body,
        out_shape=jax.ShapeDtypeStruct((M, D), jnp.float32),
        in_specs=[
            pl.BlockSpec(memory_space=pltpu.VMEM),
            pl.BlockSpec(memory_space=pl.ANY),
            pl.BlockSpec(memory_space=pltpu.VMEM),
        ],
        out_specs=pl.BlockSpec(memory_space=pltpu.VMEM),
        scratch_shapes=[
            pltpu.VMEM((M, D), jnp.float32),
            *[pltpu.VMEM((s, D), jnp.float32) for s in SIZES],
            pltpu.VMEM((256, D), jnp.float32),
            pltpu.SemaphoreType.DMA((6, 5)),
            pltpu.SemaphoreType.DMA((6, 5)),
            pltpu.SemaphoreType.DMA((6,)),
        ],
        compiler_params=pltpu.CompilerParams(collective_id=0),
    )(partial, resid, gamma2)
